# Initial kernel scaffold; baseline (speedup 1.0000x reference)
import jax
import jax.numpy as jnp
from jax.experimental import pallas as pl


def kernel(x, edge_index, edge_attr, mlp_W, mlp_b, bn_gamma, bn_beta, lin_W, lin_b, elin_W, elin_b, att, gru_Wih, gru_Whh, gru_bih, gru_bhh, out_W, out_b):
    raise NotImplementedError("write your pallas kernel here")


# speedup vs baseline: 10.5755x; 10.5755x over previous
import dataclasses
import functools

import jax
import jax.numpy as jnp
from jax import lax
from jax.experimental import pallas as pl
from jax.experimental.pallas import tpu as pltpu
from jax.experimental.pallas import tpu_sc as plsc

N = 10000
E = 320000
D_IN = 128
D = 64
NUM_STEPS = 3

NC = 2
NS = 16
NW = NC * NS
EPW = E // NW
CH = 80
NCHUNK = EPW // CH
NP = 10240
RPT = NP // NS
RZ = 128
L = 16

_f32 = jnp.float32

_SC_PARAMS = pltpu.CompilerParams(needs_layout_passes=False,
                                  use_tc_tiling_on_sc=False)


def _prep_body(x_ref, ea_ref, mlp_W_ref, mlp_b_ref, g_ref, b_ref,
               lin_W_ref, lin_b_ref, elin_W_ref, elin_b_ref, att_ref,
               h_ref, xw_ref, sd_ref, ss_ref, se_ref, bmax_ref, semax_ref):
    x = x_ref[...]
    y = lax.dot_general(x, mlp_W_ref[...], (((1,), (1,)), ((), ())),
                        preferred_element_type=_f32) + mlp_b_ref[...][None, :]
    mean = jnp.mean(y, axis=0)
    var = jnp.mean((y - mean[None, :]) ** 2, axis=0)
    scale = g_ref[...] / jnp.sqrt(var + 1e-5)
    h = jnp.maximum((y - mean[None, :]) * scale[None, :] + b_ref[...][None, :],
                    0.0)
    h_ref[...] = h
    xw = lax.dot_general(h, lin_W_ref[...], (((1,), (1,)), ((), ())),
                         preferred_element_type=_f32) + lin_b_ref[...][None, :]
    xw_ref[...] = xw
    att = att_ref[...]
    ai = att[0, 0:D]
    aj = att[0, D:2 * D]
    ae = att[0, 2 * D:3 * D]
    sd = lax.dot_general(xw, ai, (((1,), (0,)), ((), ())),
                         preferred_element_type=_f32)
    ss = lax.dot_general(xw, aj, (((1,), (0,)), ((), ())),
                         preferred_element_type=_f32)
    sd_ref[...] = sd
    ss_ref[...] = ss
    ev = lax.dot_general(elin_W_ref[...], ae, (((0,), (0,)), ((), ())),
                         preferred_element_type=_f32)
    c0 = jnp.sum(elin_b_ref[...] * ae)
    t = ea_ref[...] * ev[:, None]
    se = t[0] + t[1] + t[2] + t[3] + c0
    se_ref[...] = se
    semx = jnp.max(se)
    bd = jnp.max(sd) + jnp.max(ss) + semx
    bv = jnp.where(bd >= 0.0, bd, 0.2 * bd)
    bmax_ref[...] = jnp.full((8, 128), bv, _f32)
    semax_ref[...] = jnp.full((8, 128), semx, _f32)


def _tc_prep(x, edge_attr_t, mlp_W, mlp_b, bn_gamma, bn_beta, lin_W, lin_b,
             elin_W, elin_b, att):
    return pl.pallas_call(
        _prep_body,
        out_shape=(
            jax.ShapeDtypeStruct((N, D), _f32),
            jax.ShapeDtypeStruct((N, D), _f32),
            jax.ShapeDtypeStruct((N,), _f32),
            jax.ShapeDtypeStruct((N,), _f32),
            jax.ShapeDtypeStruct((E,), _f32),
            jax.ShapeDtypeStruct((8, 128), _f32),
            jax.ShapeDtypeStruct((8, 128), _f32),
        ),
    )(x, edge_attr_t, mlp_W, mlp_b, bn_gamma, bn_beta, lin_W, lin_b,
      elin_W, elin_b, att)


def _gru(m, h, Wih_ref, Whh_ref, bih_ref, bhh_ref):
    def mm(a, w):
        return lax.dot_general(a, w, (((1,), (1,)), ((), ())),
                               preferred_element_type=_f32)
    Wih = Wih_ref[...]
    Whh = Whh_ref[...]
    bih = bih_ref[...]
    bhh = bhh_ref[...]
    i_r = mm(m, Wih[0:D, :]) + bih[0:D][None, :]
    i_z = mm(m, Wih[D:2 * D, :]) + bih[D:2 * D][None, :]
    i_n = mm(m, Wih[2 * D:3 * D, :]) + bih[2 * D:3 * D][None, :]
    h_r = mm(h, Whh[0:D, :]) + bhh[0:D][None, :]
    h_z = mm(h, Whh[D:2 * D, :]) + bhh[D:2 * D][None, :]
    h_n = mm(h, Whh[2 * D:3 * D, :]) + bhh[2 * D:3 * D][None, :]
    r = jax.nn.sigmoid(i_r + h_r)
    z = jax.nn.sigmoid(i_z + h_z)
    n = jnp.tanh(i_n + r * h_n)
    return (1.0 - z) * n + z * h


def _msg(den_ref, agg_ref):
    den = den_ref[0] + den_ref[1] + 1e-16
    agg = agg_ref[0, 0:N, :] + agg_ref[1, 0:N, :]
    return jnp.maximum(agg / den[:, None], 0.0)


def _update_body(den_ref, agg_ref, h_ref, Wih_ref, Whh_ref, bih_ref, bhh_ref,
                 lin_W_ref, lin_b_ref, att_ref, semax_ref,
                 h_out_ref, xw_ref, sd_ref, ss_ref, bmax_ref):
    m = _msg(den_ref, agg_ref)
    h = _gru(m, h_ref[...], Wih_ref, Whh_ref, bih_ref, bhh_ref)
    h_out_ref[...] = h
    xw = lax.dot_general(h, lin_W_ref[...], (((1,), (1,)), ((), ())),
                         preferred_element_type=_f32) + lin_b_ref[...][None, :]
    xw_ref[...] = xw
    att = att_ref[...]
    ai = att[0, 0:D]
    aj = att[0, D:2 * D]
    sd = lax.dot_general(xw, ai, (((1,), (0,)), ((), ())),
                         preferred_element_type=_f32)
    ss = lax.dot_general(xw, aj, (((1,), (0,)), ((), ())),
                         preferred_element_type=_f32)
    sd_ref[...] = sd
    ss_ref[...] = ss
    bd = jnp.max(sd) + jnp.max(ss) + semax_ref[0, 0]
    bv = jnp.where(bd >= 0.0, bd, 0.2 * bd)
    bmax_ref[...] = jnp.full((8, 128), bv, _f32)


def _tc_update(den, agg, h, gru_Wih, gru_Whh, gru_bih, gru_bhh,
               lin_W, lin_b, att, semax):
    return pl.pallas_call(
        _update_body,
        out_shape=(
            jax.ShapeDtypeStruct((N, D), _f32),
            jax.ShapeDtypeStruct((N, D), _f32),
            jax.ShapeDtypeStruct((N,), _f32),
            jax.ShapeDtypeStruct((N,), _f32),
            jax.ShapeDtypeStruct((8, 128), _f32),
        ),
    )(den, agg, h, gru_Wih, gru_Whh, gru_bih, gru_bhh, lin_W, lin_b, att,
      semax)


def _final_body(den_ref, agg_ref, h_ref, Wih_ref, Whh_ref, bih_ref, bhh_ref,
                x_ref, out_W_ref, out_b_ref, y_ref):
    m = _msg(den_ref, agg_ref)
    h = _gru(m, h_ref[...], Wih_ref, Whh_ref, bih_ref, bhh_ref)
    y = lax.dot_general(x_ref[...], out_W_ref[...], (((1,), (1,)), ((), ())),
                        preferred_element_type=_f32)
    y_ref[...] = y + out_b_ref[...][None, :] + h


def _tc_final(den, agg, h, gru_Wih, gru_Whh, gru_bih, gru_bhh, x, out_W,
              out_b):
    return pl.pallas_call(
        _final_body,
        out_shape=jax.ShapeDtypeStruct((N, D), _f32),
    )(den, agg, h, gru_Wih, gru_Whh, gru_bih, gru_bhh, x, out_W, out_b)


def _sc_softmax_body(sd_h, ss_h, se_h, src_h, dst_h, bmax_h,
                     ex_h, den_h,
                     sd_t, ss_t, bt, dstb, srcb, seb, exb, zt, shared_den):
    cid = lax.axis_index("c")
    sid = lax.axis_index("s")
    wid = cid * NS + sid
    base0 = wid * EPW

    pltpu.sync_copy(sd_h, sd_t)
    pltpu.sync_copy(ss_h, ss_t)
    pltpu.sync_copy(bmax_h.at[0], bt)

    @pl.when(sid == 0)
    def _():
        @pl.loop(0, N, step=L)
        def _(i):
            zt[pl.ds(i, L)] = jnp.zeros((L,), _f32)
        pltpu.sync_copy(zt, shared_den)

    plsc.subcore_barrier()
    bv = bt[pl.ds(0, L)]

    @pl.loop(0, NCHUNK)
    def _(j):
        base = base0 + j * CH
        pltpu.sync_copy(dst_h.at[pl.ds(base, CH)], dstb.at[0])
        pltpu.sync_copy(src_h.at[pl.ds(base, CH)], srcb.at[0])
        pltpu.sync_copy(se_h.at[pl.ds(base, CH)], seb)
        for k in range(CH // L):
            di = dstb[0, pl.ds(k * L, L)]
            si = srcb[0, pl.ds(k * L, L)]
            t = (plsc.load_gather(sd_t, [di]) + plsc.load_gather(ss_t, [si])
                 + seb[pl.ds(k * L, L)])
            t = jnp.where(t >= 0.0, t, 0.2 * t)
            exb[pl.ds(k * L, L)] = jnp.exp(t - bv)
        pltpu.sync_copy(exb, ex_h.at[pl.ds(base, CH)])
        pltpu.sync_copy(exb, shared_den.at[dstb.at[0]], add=True)

    plsc.subcore_barrier()

    @pl.when(sid == 0)
    def _():
        pltpu.sync_copy(shared_den, den_h.at[cid])


def _sc_softmax(sd, ss, se, src, dst, bmax):
    mesh = plsc.VectorSubcoreMesh(core_axis_name="c", subcore_axis_name="s",
                                   num_cores=NC, num_subcores=NS)
    f = pl.kernel(
        _sc_softmax_body,
        out_type=(
            jax.ShapeDtypeStruct((E,), _f32),
            jax.ShapeDtypeStruct((NC, N), _f32),
        ),
        mesh=mesh,
        compiler_params=_SC_PARAMS,
        scratch_types=[
            pltpu.VMEM((N,), _f32),
            pltpu.VMEM((N,), _f32),
            pltpu.VMEM((128,), _f32),
            pltpu.VMEM((1, CH), jnp.int32),
            pltpu.VMEM((1, CH), jnp.int32),
            pltpu.VMEM((CH,), _f32),
            pltpu.VMEM((CH,), _f32),
            pltpu.VMEM((N,), _f32),
            pltpu.VMEM_SHARED((N,), _f32),
        ],
    )
    return f(sd, ss, se, src, dst, bmax)


def _sc_agg_body(xw_h, ex_h, src_h, dst_h, agg_h,
                 srcb, dstb, exb, rows, z2, shared_agg):
    cid = lax.axis_index("c")
    sid = lax.axis_index("s")
    wid = cid * NS + sid
    base0 = wid * EPW

    @pl.loop(0, RZ)
    def _(i):
        for q in range(D // L):
            z2[i, pl.ds(q * L, L)] = jnp.zeros((L,), _f32)
    for q in range(RPT // RZ):
        pltpu.sync_copy(z2, shared_agg.at[pl.ds(sid * RPT + q * RZ, RZ)])

    plsc.subcore_barrier()

    @pl.loop(0, NCHUNK)
    def _(j):
        base = base0 + j * CH
        pltpu.sync_copy(src_h.at[pl.ds(base, CH)], srcb.at[0])
        pltpu.sync_copy(dst_h.at[pl.ds(base, CH)], dstb.at[0])
        pltpu.sync_copy(ex_h.at[pl.ds(base, CH)], exb)
        pltpu.sync_copy(xw_h.at[srcb.at[0]], rows)

        @pl.loop(0, CH)
        def _(e):
            a = plsc.load_gather(exb, [jnp.full((L,), e, jnp.int32)])
            for q in range(D // L):
                rows[e, pl.ds(q * L, L)] = rows[e, pl.ds(q * L, L)] * a

        pltpu.sync_copy(rows, shared_agg.at[dstb.at[0]], add=True)

    plsc.subcore_barrier()
    for q in range(RPT // RZ):
        r0 = sid * RPT + q * RZ
        pltpu.sync_copy(shared_agg.at[pl.ds(r0, RZ)],
                        agg_h.at[cid, pl.ds(r0, RZ)])


def _sc_agg(xw, ex, src, dst):
    mesh = plsc.VectorSubcoreMesh(core_axis_name="c", subcore_axis_name="s",
                                   num_cores=NC, num_subcores=NS)
    f = pl.kernel(
        _sc_agg_body,
        out_type=jax.ShapeDtypeStruct((NC, NP, D), _f32),
        mesh=mesh,
        compiler_params=_SC_PARAMS,
        scratch_types=[
            pltpu.VMEM((1, CH), jnp.int32),
            pltpu.VMEM((1, CH), jnp.int32),
            pltpu.VMEM((CH,), _f32),
            pltpu.VMEM((CH, D), _f32),
            pltpu.VMEM((RZ, D), _f32),
            pltpu.VMEM_SHARED((NP, D), _f32),
        ],
    )
    return f(xw, ex, src, dst)


def kernel(x, edge_index, edge_attr, mlp_W, mlp_b, bn_gamma, bn_beta,
           lin_W, lin_b, elin_W, elin_b, att, gru_Wih, gru_Whh,
           gru_bih, gru_bhh, out_W, out_b):
    src = edge_index[0]
    dst = edge_index[1]

    h, xw, sd, ss, se, bmax, semax = _tc_prep(
        x, edge_attr.T, mlp_W, mlp_b, bn_gamma, bn_beta, lin_W, lin_b,
        elin_W, elin_b, att)
    for step in range(NUM_STEPS):
        ex, den = _sc_softmax(sd, ss, se, src, dst, bmax)
        agg = _sc_agg(xw, ex, src, dst)
        if step < NUM_STEPS - 1:
            h, xw, sd, ss, bmax = _tc_update(
                den, agg, h, gru_Wih, gru_Whh, gru_bih, gru_bhh,
                lin_W, lin_b, att, semax)
        else:
            out = _tc_final(den, agg, h, gru_Wih, gru_Whh, gru_bih,
                            gru_bhh, x, out_W, out_b)
    return out


# speedup vs baseline: 30.0633x; 2.8427x over previous
import dataclasses
import functools

import jax
import jax.numpy as jnp
from jax import lax
from jax.experimental import pallas as pl
from jax.experimental.pallas import tpu as pltpu
from jax.experimental.pallas import tpu_sc as plsc

N = 10000
E = 320000
D_IN = 128
D = 64
NUM_STEPS = 3

NC = 2
NS = 16
NW = NC * NS
EPW = E // NW
CH = 80
NCHUNK = EPW // CH
NP = 10240
RPT = NP // NS
RZ = 128
L = 16

_f32 = jnp.float32

_SC_PARAMS = pltpu.CompilerParams(needs_layout_passes=False,
                                  use_tc_tiling_on_sc=False)


def _prep_body(x_ref, ea_ref, mlp_W_ref, mlp_b_ref, g_ref, b_ref,
               lin_W_ref, lin_b_ref, elin_W_ref, elin_b_ref, att_ref,
               h_ref, xw_ref, sd_ref, ss_ref, se_ref, bmax_ref, semax_ref):
    x = x_ref[...]
    y = lax.dot_general(x, mlp_W_ref[...], (((1,), (1,)), ((), ())),
                        preferred_element_type=_f32) + mlp_b_ref[...][None, :]
    mean = jnp.mean(y, axis=0)
    var = jnp.mean((y - mean[None, :]) ** 2, axis=0)
    scale = g_ref[...] / jnp.sqrt(var + 1e-5)
    h = jnp.maximum((y - mean[None, :]) * scale[None, :] + b_ref[...][None, :],
                    0.0)
    h_ref[...] = h
    xw = lax.dot_general(h, lin_W_ref[...], (((1,), (1,)), ((), ())),
                         preferred_element_type=_f32) + lin_b_ref[...][None, :]
    xw_ref[...] = xw
    att = att_ref[...]
    ai = att[0, 0:D]
    aj = att[0, D:2 * D]
    ae = att[0, 2 * D:3 * D]
    sd = lax.dot_general(xw, ai, (((1,), (0,)), ((), ())),
                         preferred_element_type=_f32)
    ss = lax.dot_general(xw, aj, (((1,), (0,)), ((), ())),
                         preferred_element_type=_f32)
    sd_ref[...] = sd
    ss_ref[...] = ss
    ev = lax.dot_general(elin_W_ref[...], ae, (((0,), (0,)), ((), ())),
                         preferred_element_type=_f32)
    c0 = jnp.sum(elin_b_ref[...] * ae)
    t = ea_ref[...] * ev[:, None]
    se = t[0] + t[1] + t[2] + t[3] + c0
    se_ref[...] = se
    semx = jnp.max(se)
    bd = jnp.max(sd) + jnp.max(ss) + semx
    bv = jnp.where(bd >= 0.0, bd, 0.2 * bd)
    bmax_ref[...] = jnp.full((8, 128), bv, _f32)
    semax_ref[...] = jnp.full((8, 128), semx, _f32)


def _tc_prep(x, edge_attr_t, mlp_W, mlp_b, bn_gamma, bn_beta, lin_W, lin_b,
             elin_W, elin_b, att):
    return pl.pallas_call(
        _prep_body,
        out_shape=(
            jax.ShapeDtypeStruct((N, D), _f32),
            jax.ShapeDtypeStruct((N, D), _f32),
            jax.ShapeDtypeStruct((N,), _f32),
            jax.ShapeDtypeStruct((N,), _f32),
            jax.ShapeDtypeStruct((E,), _f32),
            jax.ShapeDtypeStruct((8, 128), _f32),
            jax.ShapeDtypeStruct((8, 128), _f32),
        ),
    )(x, edge_attr_t, mlp_W, mlp_b, bn_gamma, bn_beta, lin_W, lin_b,
      elin_W, elin_b, att)


def _gru(m, h, Wih_ref, Whh_ref, bih_ref, bhh_ref):
    def mm(a, w):
        return lax.dot_general(a, w, (((1,), (1,)), ((), ())),
                               preferred_element_type=_f32)
    Wih = Wih_ref[...]
    Whh = Whh_ref[...]
    bih = bih_ref[...]
    bhh = bhh_ref[...]
    i_r = mm(m, Wih[0:D, :]) + bih[0:D][None, :]
    i_z = mm(m, Wih[D:2 * D, :]) + bih[D:2 * D][None, :]
    i_n = mm(m, Wih[2 * D:3 * D, :]) + bih[2 * D:3 * D][None, :]
    h_r = mm(h, Whh[0:D, :]) + bhh[0:D][None, :]
    h_z = mm(h, Whh[D:2 * D, :]) + bhh[D:2 * D][None, :]
    h_n = mm(h, Whh[2 * D:3 * D, :]) + bhh[2 * D:3 * D][None, :]
    r = jax.nn.sigmoid(i_r + h_r)
    z = jax.nn.sigmoid(i_z + h_z)
    n = jnp.tanh(i_n + r * h_n)
    return (1.0 - z) * n + z * h


def _msg(den_ref, agg_ref):
    den = den_ref[0] + den_ref[1] + 1e-16
    agg = agg_ref[0, 0:N, :] + agg_ref[1, 0:N, :]
    return jnp.maximum(agg / den[:, None], 0.0)


def _update_body(den_ref, agg_ref, h_ref, Wih_ref, Whh_ref, bih_ref, bhh_ref,
                 lin_W_ref, lin_b_ref, att_ref, semax_ref,
                 h_out_ref, xw_ref, sd_ref, ss_ref, bmax_ref):
    m = _msg(den_ref, agg_ref)
    h = _gru(m, h_ref[...], Wih_ref, Whh_ref, bih_ref, bhh_ref)
    h_out_ref[...] = h
    xw = lax.dot_general(h, lin_W_ref[...], (((1,), (1,)), ((), ())),
                         preferred_element_type=_f32) + lin_b_ref[...][None, :]
    xw_ref[...] = xw
    att = att_ref[...]
    ai = att[0, 0:D]
    aj = att[0, D:2 * D]
    sd = lax.dot_general(xw, ai, (((1,), (0,)), ((), ())),
                         preferred_element_type=_f32)
    ss = lax.dot_general(xw, aj, (((1,), (0,)), ((), ())),
                         preferred_element_type=_f32)
    sd_ref[...] = sd
    ss_ref[...] = ss
    bd = jnp.max(sd) + jnp.max(ss) + semax_ref[0, 0]
    bv = jnp.where(bd >= 0.0, bd, 0.2 * bd)
    bmax_ref[...] = jnp.full((8, 128), bv, _f32)


def _tc_update(den, agg, h, gru_Wih, gru_Whh, gru_bih, gru_bhh,
               lin_W, lin_b, att, semax):
    return pl.pallas_call(
        _update_body,
        out_shape=(
            jax.ShapeDtypeStruct((N, D), _f32),
            jax.ShapeDtypeStruct((N, D), _f32),
            jax.ShapeDtypeStruct((N,), _f32),
            jax.ShapeDtypeStruct((N,), _f32),
            jax.ShapeDtypeStruct((8, 128), _f32),
        ),
    )(den, agg, h, gru_Wih, gru_Whh, gru_bih, gru_bhh, lin_W, lin_b, att,
      semax)


def _final_body(den_ref, agg_ref, h_ref, Wih_ref, Whh_ref, bih_ref, bhh_ref,
                x_ref, out_W_ref, out_b_ref, y_ref):
    m = _msg(den_ref, agg_ref)
    h = _gru(m, h_ref[...], Wih_ref, Whh_ref, bih_ref, bhh_ref)
    y = lax.dot_general(x_ref[...], out_W_ref[...], (((1,), (1,)), ((), ())),
                        preferred_element_type=_f32)
    y_ref[...] = y + out_b_ref[...][None, :] + h


def _tc_final(den, agg, h, gru_Wih, gru_Whh, gru_bih, gru_bhh, x, out_W,
              out_b):
    return pl.pallas_call(
        _final_body,
        out_shape=jax.ShapeDtypeStruct((N, D), _f32),
    )(den, agg, h, gru_Wih, gru_Whh, gru_bih, gru_bhh, x, out_W, out_b)


SUP = 25
NMAC = NCHUNK // SUP


def _sc_step_body(sd_h, ss_h, se_h, src_h, dst_h, bmax_h, xw_h,
                  den_h, agg_h,
                  sd_t, ss_t, bt, srcb, dstb, seb, exb, rows, z2, zt,
                  shared_den, shared_agg, sem0, sem1):
    cid = lax.axis_index("c")
    sid = lax.axis_index("s")
    wid = cid * NS + sid
    row0 = wid * NCHUNK

    pltpu.sync_copy(sd_h, sd_t)
    pltpu.sync_copy(ss_h, ss_t)
    pltpu.sync_copy(bmax_h.at[0], bt)

    @pl.when(sid == 0)
    def _():
        @pl.loop(0, N, step=L)
        def _(i):
            zt[pl.ds(i, L)] = jnp.zeros((L,), _f32)
        pltpu.sync_copy(zt, shared_den)

    @pl.loop(0, RZ)
    def _(i):
        for q in range(D // L):
            z2[i, pl.ds(q * L, L)] = jnp.zeros((L,), _f32)
    for q in range(RPT // RZ):
        pltpu.sync_copy(z2, shared_agg.at[pl.ds(sid * RPT + q * RZ, RZ)])

    plsc.subcore_barrier()
    bv = bt[pl.ds(0, L)]

    def issue(c, buf, sem):
        return pltpu.async_copy(xw_h.at[srcb.at[c]], buf, sem)

    def process(c, buf, sem):
        pltpu.make_async_copy(xw_h.at[srcb.at[c]], buf, sem).wait()
        for k in range(CH // L):
            di = dstb[c, pl.ds(k * L, L)]
            si = srcb[c, pl.ds(k * L, L)]
            t = (plsc.load_gather(sd_t, [di]) + plsc.load_gather(ss_t, [si])
                 + seb[c, pl.ds(k * L, L)])
            t = jnp.where(t >= 0.0, t, 0.2 * t)
            exb[c, pl.ds(k * L, L)] = jnp.exp(t - bv)

        cfull = jnp.full((L,), c, jnp.int32)

        @pl.loop(0, CH)
        def _(e):
            a = plsc.load_gather(exb, [cfull, jnp.full((L,), e, jnp.int32)])
            for q in range(D // L):
                buf[e, pl.ds(q * L, L)] = buf[e, pl.ds(q * L, L)] * a

        pltpu.sync_copy(buf, shared_agg.at[dstb.at[c]], add=True)
        pltpu.sync_copy(exb.at[c], shared_den.at[dstb.at[c]], add=True)

    @pl.loop(0, NMAC)
    def _(m):
        r0 = row0 + m * SUP
        pltpu.sync_copy(src_h.at[pl.ds(r0, SUP)], srcb)
        pltpu.sync_copy(dst_h.at[pl.ds(r0, SUP)], dstb)
        pltpu.sync_copy(se_h.at[pl.ds(r0, SUP)], seb)
        issue(0, rows.at[0], sem0)

        @pl.loop(0, SUP - 1, step=2)
        def _(c):
            issue(c + 1, rows.at[1], sem1)
            process(c, rows.at[0], sem0)
            issue(c + 2, rows.at[0], sem0)
            process(c + 1, rows.at[1], sem1)

        process(SUP - 1, rows.at[0], sem0)

    plsc.subcore_barrier()

    @pl.when(sid == 0)
    def _():
        pltpu.sync_copy(shared_den, den_h.at[cid])
    for q in range(RPT // RZ):
        r0 = sid * RPT + q * RZ
        pltpu.sync_copy(shared_agg.at[pl.ds(r0, RZ)],
                        agg_h.at[cid, pl.ds(r0, RZ)])


def _sc_step(sd, ss, se2, src2, dst2, bmax, xw):
    mesh = plsc.VectorSubcoreMesh(core_axis_name="c", subcore_axis_name="s",
                                   num_cores=NC, num_subcores=NS)
    f = pl.kernel(
        _sc_step_body,
        out_type=(
            jax.ShapeDtypeStruct((NC, N), _f32),
            jax.ShapeDtypeStruct((NC, NP, D), _f32),
        ),
        mesh=mesh,
        compiler_params=_SC_PARAMS,
        scratch_types=[
            pltpu.VMEM((N,), _f32),
            pltpu.VMEM((N,), _f32),
            pltpu.VMEM((128,), _f32),
            pltpu.VMEM((SUP, CH), jnp.int32),
            pltpu.VMEM((SUP, CH), jnp.int32),
            pltpu.VMEM((SUP, CH), _f32),
            pltpu.VMEM((SUP, CH), _f32),
            pltpu.VMEM((2, CH, D), _f32),
            pltpu.VMEM((RZ, D), _f32),
            pltpu.VMEM((N,), _f32),
            pltpu.VMEM_SHARED((N,), _f32),
            pltpu.VMEM_SHARED((NP, D), _f32),
            pltpu.SemaphoreType.DMA,
            pltpu.SemaphoreType.DMA,
        ],
    )
    return f(sd, ss, se2, src2, dst2, bmax, xw)


def kernel(x, edge_index, edge_attr, mlp_W, mlp_b, bn_gamma, bn_beta,
           lin_W, lin_b, elin_W, elin_b, att, gru_Wih, gru_Whh,
           gru_bih, gru_bhh, out_W, out_b):
    src2 = edge_index[0].reshape(E // CH, CH)
    dst2 = edge_index[1].reshape(E // CH, CH)

    h, xw, sd, ss, se, bmax, semax = _tc_prep(
        x, edge_attr.T, mlp_W, mlp_b, bn_gamma, bn_beta, lin_W, lin_b,
        elin_W, elin_b, att)
    se2 = se.reshape(E // CH, CH)
    for step in range(NUM_STEPS):
        den, agg = _sc_step(sd, ss, se2, src2, dst2, bmax, xw)
        if step < NUM_STEPS - 1:
            h, xw, sd, ss, bmax = _tc_update(
                den, agg, h, gru_Wih, gru_Whh, gru_bih, gru_bhh,
                lin_W, lin_b, att, semax)
        else:
            out = _tc_final(den, agg, h, gru_Wih, gru_Whh, gru_bih,
                            gru_bhh, x, out_W, out_b)
    return out


# speedup vs baseline: 44.2930x; 1.4733x over previous
import dataclasses
import functools

import jax
import jax.numpy as jnp
from jax import lax
from jax.experimental import pallas as pl
from jax.experimental.pallas import tpu as pltpu
from jax.experimental.pallas import tpu_sc as plsc

N = 10000
E = 320000
D_IN = 128
D = 64
NUM_STEPS = 3

NC = 2
NS = 16
NW = NC * NS
EPW = E // NW
CH = 80
NCHUNK = EPW // CH
NP = 10240
RPT = NP // NS
RZ = 128
L = 16

_f32 = jnp.float32

_SC_PARAMS = pltpu.CompilerParams(needs_layout_passes=False,
                                  use_tc_tiling_on_sc=False)


def _prep_body(x_ref, ea_ref, mlp_W_ref, mlp_b_ref, g_ref, b_ref,
               lin_W_ref, lin_b_ref, elin_W_ref, elin_b_ref, att_ref,
               h_ref, xw_ref, sd_ref, ss_ref, se_ref, bmax_ref, semax_ref):
    x = x_ref[...]
    y = lax.dot_general(x, mlp_W_ref[...], (((1,), (1,)), ((), ())),
                        preferred_element_type=_f32) + mlp_b_ref[...][None, :]
    mean = jnp.mean(y, axis=0)
    var = jnp.mean((y - mean[None, :]) ** 2, axis=0)
    scale = g_ref[...] / jnp.sqrt(var + 1e-5)
    h = jnp.maximum((y - mean[None, :]) * scale[None, :] + b_ref[...][None, :],
                    0.0)
    h_ref[...] = h
    xw = lax.dot_general(h, lin_W_ref[...], (((1,), (1,)), ((), ())),
                         preferred_element_type=_f32) + lin_b_ref[...][None, :]
    xw_ref[...] = xw
    att = att_ref[...]
    ai = att[0, 0:D]
    aj = att[0, D:2 * D]
    ae = att[0, 2 * D:3 * D]
    sd = lax.dot_general(xw, ai, (((1,), (0,)), ((), ())),
                         preferred_element_type=_f32)
    ss = lax.dot_general(xw, aj, (((1,), (0,)), ((), ())),
                         preferred_element_type=_f32)
    sd_ref[...] = sd
    ss_ref[...] = ss
    ev = lax.dot_general(elin_W_ref[...], ae, (((0,), (0,)), ((), ())),
                         preferred_element_type=_f32)
    c0 = jnp.sum(elin_b_ref[...] * ae)
    t = ea_ref[...] * ev[:, None]
    se = t[0] + t[1] + t[2] + t[3] + c0
    se_ref[...] = se
    semx = jnp.max(se)
    bd = jnp.max(sd) + jnp.max(ss) + semx
    bv = jnp.where(bd >= 0.0, bd, 0.2 * bd)
    bmax_ref[...] = jnp.full((8, 128), bv, _f32)
    semax_ref[...] = jnp.full((8, 128), semx, _f32)


def _tc_prep(x, edge_attr_t, mlp_W, mlp_b, bn_gamma, bn_beta, lin_W, lin_b,
             elin_W, elin_b, att):
    return pl.pallas_call(
        _prep_body,
        out_shape=(
            jax.ShapeDtypeStruct((N, D), _f32),
            jax.ShapeDtypeStruct((N, D), _f32),
            jax.ShapeDtypeStruct((N,), _f32),
            jax.ShapeDtypeStruct((N,), _f32),
            jax.ShapeDtypeStruct((E,), _f32),
            jax.ShapeDtypeStruct((8, 128), _f32),
            jax.ShapeDtypeStruct((8, 128), _f32),
        ),
    )(x, edge_attr_t, mlp_W, mlp_b, bn_gamma, bn_beta, lin_W, lin_b,
      elin_W, elin_b, att)


def _gru(m, h, Wih_ref, Whh_ref, bih_ref, bhh_ref):
    def mm(a, w):
        return lax.dot_general(a, w, (((1,), (1,)), ((), ())),
                               preferred_element_type=_f32)
    Wih = Wih_ref[...]
    Whh = Whh_ref[...]
    bih = bih_ref[...]
    bhh = bhh_ref[...]
    i_r = mm(m, Wih[0:D, :]) + bih[0:D][None, :]
    i_z = mm(m, Wih[D:2 * D, :]) + bih[D:2 * D][None, :]
    i_n = mm(m, Wih[2 * D:3 * D, :]) + bih[2 * D:3 * D][None, :]
    h_r = mm(h, Whh[0:D, :]) + bhh[0:D][None, :]
    h_z = mm(h, Whh[D:2 * D, :]) + bhh[D:2 * D][None, :]
    h_n = mm(h, Whh[2 * D:3 * D, :]) + bhh[2 * D:3 * D][None, :]
    r = jax.nn.sigmoid(i_r + h_r)
    z = jax.nn.sigmoid(i_z + h_z)
    n = jnp.tanh(i_n + r * h_n)
    return (1.0 - z) * n + z * h


def _msg(den_ref, agg_ref):
    den = den_ref[0] + den_ref[1] + 1e-16
    agg = agg_ref[0, 0:N, :] + agg_ref[1, 0:N, :]
    return jnp.maximum(agg / den[:, None], 0.0)


def _update_body(den_ref, agg_ref, h_ref, Wih_ref, Whh_ref, bih_ref, bhh_ref,
                 lin_W_ref, lin_b_ref, att_ref, semax_ref,
                 h_out_ref, xw_ref, sd_ref, ss_ref, bmax_ref):
    m = _msg(den_ref, agg_ref)
    h = _gru(m, h_ref[...], Wih_ref, Whh_ref, bih_ref, bhh_ref)
    h_out_ref[...] = h
    xw = lax.dot_general(h, lin_W_ref[...], (((1,), (1,)), ((), ())),
                         preferred_element_type=_f32) + lin_b_ref[...][None, :]
    xw_ref[...] = xw
    att = att_ref[...]
    ai = att[0, 0:D]
    aj = att[0, D:2 * D]
    sd = lax.dot_general(xw, ai, (((1,), (0,)), ((), ())),
                         preferred_element_type=_f32)
    ss = lax.dot_general(xw, aj, (((1,), (0,)), ((), ())),
                         preferred_element_type=_f32)
    sd_ref[...] = sd
    ss_ref[...] = ss
    bd = jnp.max(sd) + jnp.max(ss) + semax_ref[0, 0]
    bv = jnp.where(bd >= 0.0, bd, 0.2 * bd)
    bmax_ref[...] = jnp.full((8, 128), bv, _f32)


def _tc_update(den, agg, h, gru_Wih, gru_Whh, gru_bih, gru_bhh,
               lin_W, lin_b, att, semax):
    return pl.pallas_call(
        _update_body,
        out_shape=(
            jax.ShapeDtypeStruct((N, D), _f32),
            jax.ShapeDtypeStruct((N, D), _f32),
            jax.ShapeDtypeStruct((N,), _f32),
            jax.ShapeDtypeStruct((N,), _f32),
            jax.ShapeDtypeStruct((8, 128), _f32),
        ),
    )(den, agg, h, gru_Wih, gru_Whh, gru_bih, gru_bhh, lin_W, lin_b, att,
      semax)


def _final_body(den_ref, agg_ref, h_ref, Wih_ref, Whh_ref, bih_ref, bhh_ref,
                x_ref, out_W_ref, out_b_ref, y_ref):
    m = _msg(den_ref, agg_ref)
    h = _gru(m, h_ref[...], Wih_ref, Whh_ref, bih_ref, bhh_ref)
    y = lax.dot_general(x_ref[...], out_W_ref[...], (((1,), (1,)), ((), ())),
                        preferred_element_type=_f32)
    y_ref[...] = y + out_b_ref[...][None, :] + h


def _tc_final(den, agg, h, gru_Wih, gru_Whh, gru_bih, gru_bhh, x, out_W,
              out_b):
    return pl.pallas_call(
        _final_body,
        out_shape=jax.ShapeDtypeStruct((N, D), _f32),
    )(den, agg, h, gru_Wih, gru_Whh, gru_bih, gru_bhh, x, out_W, out_b)


SUP = 25
NMAC = NCHUNK // SUP


def _sc_step_body(sd_h, ss_h, se_h, src_h, dst_h, bmax_h, xw_h,
                  den_h, agg_h,
                  sd_t, ss_t, bt, srcb, dstb, seb, exb, rows, z2, zt,
                  shared_den, shared_agg, sg0, sg1, sg2, sg3,
                  sa0, sa1, sa2, sa3, sx):
    cid = lax.axis_index("c")
    sid = lax.axis_index("s")
    wid = cid * NS + sid
    row0 = wid * NCHUNK

    pltpu.sync_copy(sd_h, sd_t)
    pltpu.sync_copy(ss_h, ss_t)
    pltpu.sync_copy(bmax_h.at[0], bt)

    @pl.when(sid == 0)
    def _():
        @pl.loop(0, N, step=L)
        def _(i):
            zt[pl.ds(i, L)] = jnp.zeros((L,), _f32)
        pltpu.sync_copy(zt, shared_den)

    @pl.loop(0, RZ)
    def _(i):
        for q in range(D // L):
            z2[i, pl.ds(q * L, L)] = jnp.zeros((L,), _f32)
    for q in range(RPT // RZ):
        pltpu.sync_copy(z2, shared_agg.at[pl.ds(sid * RPT + q * RZ, RZ)])

    plsc.subcore_barrier()
    bv = bt[pl.ds(0, L)]

    def issue_g(c, buf, sem):
        pltpu.async_copy(xw_h.at[srcb.at[c]], buf, sem)

    def wait_g(c, buf, sem):
        pltpu.make_async_copy(xw_h.at[srcb.at[c]], buf, sem).wait()

    def issue_s(c, buf, sem):
        pltpu.async_copy(buf, shared_agg.at[dstb.at[c]], sem, add=True)
        pltpu.async_copy(exb.at[c], shared_den.at[dstb.at[c]], sx, add=True)

    def wait_s(c, buf, sem):
        pltpu.make_async_copy(buf, shared_agg.at[dstb.at[c]], sem).wait()

    def compute(c, buf):
        for k in range(CH // L):
            di = dstb[c, pl.ds(k * L, L)]
            si = srcb[c, pl.ds(k * L, L)]
            t = (plsc.load_gather(sd_t, [di]) + plsc.load_gather(ss_t, [si])
                 + seb[c, pl.ds(k * L, L)])
            t = jnp.where(t >= 0.0, t, 0.2 * t)
            exb[c, pl.ds(k * L, L)] = jnp.exp(t - bv)

        cfull = jnp.full((L,), c, jnp.int32)

        @plsc.parallel_loop(0, CH, 1, unroll=2)
        def _(e):
            a = plsc.load_gather(exb, [cfull, jnp.full((L,), e, jnp.int32)])
            for q in range(D // L):
                buf[e, pl.ds(q * L, L)] = buf[e, pl.ds(q * L, L)] * a

    bufs = [rows.at[0], rows.at[1], rows.at[2], rows.at[3]]
    sgs = [sg0, sg1, sg2, sg3]
    sas = [sa0, sa1, sa2, sa3]

    @pl.loop(0, NMAC)
    def _(m):
        r0 = row0 + m * SUP
        pltpu.sync_copy(src_h.at[pl.ds(r0, SUP)], srcb)
        pltpu.sync_copy(dst_h.at[pl.ds(r0, SUP)], dstb)
        pltpu.sync_copy(se_h.at[pl.ds(r0, SUP)], seb)
        for j in range(3):
            issue_g(j, bufs[j], sgs[j])

        @pl.loop(0, SUP - 1, step=4)
        def _(c):
            for j in range(4):
                q = c + j
                wait_g(q, bufs[j], sgs[j])
                compute(q, bufs[j])
                issue_s(q, bufs[j], sas[j])
                jp = (j + 3) % 4
                if j == 0:
                    @pl.when(c > 0)
                    def _():
                        wait_s(q - 1, bufs[jp], sas[jp])
                else:
                    wait_s(q - 1, bufs[jp], sas[jp])
                if j >= 2:
                    @pl.when(q + 3 < SUP)
                    def _():
                        issue_g(q + 3, bufs[jp], sgs[jp])
                else:
                    issue_g(q + 3, bufs[jp], sgs[jp])

        q = SUP - 1
        wait_g(q, bufs[0], sgs[0])
        compute(q, bufs[0])
        issue_s(q, bufs[0], sas[0])
        wait_s(q - 1, bufs[3], sas[3])
        wait_s(q, bufs[0], sas[0])

        @pl.loop(0, SUP)
        def _(c):
            pltpu.make_async_copy(exb.at[c], shared_den.at[dstb.at[c]],
                                  sx).wait()

    plsc.subcore_barrier()

    @pl.when(sid == 0)
    def _():
        pltpu.sync_copy(shared_den, den_h.at[cid])
    for q in range(RPT // RZ):
        r0 = sid * RPT + q * RZ
        pltpu.sync_copy(shared_agg.at[pl.ds(r0, RZ)],
                        agg_h.at[cid, pl.ds(r0, RZ)])


def _sc_step(sd, ss, se2, src2, dst2, bmax, xw):
    mesh = plsc.VectorSubcoreMesh(core_axis_name="c", subcore_axis_name="s",
                                   num_cores=NC, num_subcores=NS)
    f = pl.kernel(
        _sc_step_body,
        out_type=(
            jax.ShapeDtypeStruct((NC, N), _f32),
            jax.ShapeDtypeStruct((NC, NP, D), _f32),
        ),
        mesh=mesh,
        compiler_params=_SC_PARAMS,
        scratch_types=[
            pltpu.VMEM((N,), _f32),
            pltpu.VMEM((N,), _f32),
            pltpu.VMEM((128,), _f32),
            pltpu.VMEM((SUP, CH), jnp.int32),
            pltpu.VMEM((SUP, CH), jnp.int32),
            pltpu.VMEM((SUP, CH), _f32),
            pltpu.VMEM((SUP, CH), _f32),
            pltpu.VMEM((4, CH, D), _f32),
            pltpu.VMEM((RZ, D), _f32),
            pltpu.VMEM((N,), _f32),
            pltpu.VMEM_SHARED((N,), _f32),
            pltpu.VMEM_SHARED((NP, D), _f32),
        ] + [pltpu.SemaphoreType.DMA] * 9,
    )
    return f(sd, ss, se2, src2, dst2, bmax, xw)


def kernel(x, edge_index, edge_attr, mlp_W, mlp_b, bn_gamma, bn_beta,
           lin_W, lin_b, elin_W, elin_b, att, gru_Wih, gru_Whh,
           gru_bih, gru_bhh, out_W, out_b):
    src2 = edge_index[0].reshape(E // CH, CH)
    dst2 = edge_index[1].reshape(E // CH, CH)

    h, xw, sd, ss, se, bmax, semax = _tc_prep(
        x, edge_attr.T, mlp_W, mlp_b, bn_gamma, bn_beta, lin_W, lin_b,
        elin_W, elin_b, att)
    se2 = se.reshape(E // CH, CH)
    for step in range(NUM_STEPS):
        den, agg = _sc_step(sd, ss, se2, src2, dst2, bmax, xw)
        if step < NUM_STEPS - 1:
            h, xw, sd, ss, bmax = _tc_update(
                den, agg, h, gru_Wih, gru_Whh, gru_bih, gru_bhh,
                lin_W, lin_b, att, semax)
        else:
            out = _tc_final(den, agg, h, gru_Wih, gru_Whh, gru_bih,
                            gru_bhh, x, out_W, out_b)
    return out
